# baseline (speedup 1.0000x reference)
import jax
import jax.numpy as jnp
from jax import lax
from jax.experimental import pallas as pl
from jax.experimental.pallas import tpu as pltpu
from jax.experimental.pallas import tpu_sc as plsc

_T = 16384
_E = 64
_NC, _NS, _L = 2, 16, 16
_NW = _NC * _NS
_RPW = _T // _NW
_H = _RPW // 2
_GH = _H // _L
_CSB = 8


def _sc_body(qt_hbm, hist_hbm, chunk, obuf, sem0, sem1):
    wid = lax.axis_index("s") * _NC + lax.axis_index("c")
    base = wid * _RPW
    cp0 = pltpu.async_copy(
        qt_hbm.at[:, pl.ds(base, _H)], chunk.at[:, pl.ds(0, _H)], sem0)
    cp1 = pltpu.async_copy(
        qt_hbm.at[:, pl.ds(base + _H, _H)], chunk.at[:, pl.ds(_H, _H)], sem1)

    zf = jnp.zeros((_L,), jnp.float32)
    ones = jnp.ones((_L,), jnp.float32)
    for j in range(_E // _L):
        obuf[pl.ds(j * _L, _L)] = zf

    def group_argmax(g):
        v = [chunk[e, pl.ds(g * _L, _L)] for e in range(_E)]
        mt = list(v)
        at = [jnp.full((_L,), e, jnp.int32) for e in range(_E)]
        n = 1
        while n < _E:
            for j in range(0, _E, 2 * n):
                ge = mt[j] >= mt[j + n]
                mt[j] = jnp.where(ge, mt[j], mt[j + n])
                at[j] = jnp.where(ge, at[j], at[j + n])
            n *= 2
        plsc.addupdate_scatter(obuf, [at[0]], ones)

    def g_body(i, carry):
        group_argmax(2 * i)
        group_argmax(2 * i + 1)
        return carry

    cp0.wait()
    lax.fori_loop(0, _GH // 2, g_body, 0)
    cp1.wait()
    lax.fori_loop(_GH // 2, _GH, g_body, 0)

    pltpu.sync_copy(obuf, hist_hbm.at[wid])


_sc_call = pl.kernel(
    _sc_body,
    out_type=jax.ShapeDtypeStruct((_NW, _E), jnp.float32),
    mesh=plsc.VectorSubcoreMesh(core_axis_name="c", subcore_axis_name="s"),
    compiler_params=pltpu.CompilerParams(needs_layout_passes=False),
    scratch_types=[
        pltpu.VMEM((_E, _RPW), jnp.float32),
        pltpu.VMEM((_E,), jnp.float32),
        pltpu.SemaphoreType.DMA,
        pltpu.SemaphoreType.DMA,
    ],
)


def _cs_body(qt_ref, cs_ref):
    v = qt_ref[...].reshape(_E, _T // 128, 128)
    cs_ref[...] = jnp.sum(jnp.sum(v, axis=1), axis=1).reshape(1, _E)


def _fin_body(cs_ref, h_ref, o_ref):
    ct = jnp.sum(h_ref[...], axis=0)
    o_ref[...] = (jnp.sum(cs_ref[0, :] * ct) * (_E / (_T * _T))).reshape(1, 1)


def kernel(q):
    qt = q.T
    hist = _sc_call(qt)
    cs = pl.pallas_call(
        _cs_body,
        out_shape=jax.ShapeDtypeStruct((1, _E), jnp.float32),
    )(qt)
    out = pl.pallas_call(
        _fin_body,
        out_shape=jax.ShapeDtypeStruct((1, 1), jnp.float32),
    )(cs, hist)
    return out[0, 0]
